# Initial kernel scaffold; baseline (speedup 1.0000x reference)
import functools

import numpy as np
import jax
import jax.numpy as jnp
from jax import lax
from jax.experimental import pallas as pl
from jax.experimental.pallas import tpu as pltpu
from jax.experimental.pallas import tpu_sc as plsc

_MAX_LEN = 200
_EMB = 64
_BATCH = 4096

_NC = 2
_NS = 16
_NW = _NC * _NS
_TOTAL = _BATCH * _MAX_LEN
_PER_W = _TOTAL // _NW
_CHUNK = 128
_N_CHUNKS = _PER_W // _CHUNK
_LANES = 16


def _pos_encoding():
    pos = np.arange(_MAX_LEN).reshape(-1, 1)
    i = np.arange(_EMB / 2).reshape(1, -1)
    pe = np.empty((_MAX_LEN, _EMB))
    pe[:, 0::2] = np.sin(pos / np.power(10000, 2 * i / _EMB))
    pe[:, 1::2] = np.cos(pos / np.power(10000, 2 * i / _EMB))
    return pe.astype(np.float32)


_POS2X = np.concatenate([_pos_encoding(), _pos_encoding()], axis=0)


def _body(idx_hbm, table_hbm, pos_hbm, emb_out, mask_out,
          idx_v, pos_v, rows_v, mask_v, gsem):
    c = lax.axis_index("c")
    s = lax.axis_index("s")
    wid = s * _NC + c
    base = wid * _PER_W

    pltpu.sync_copy(idx_hbm.at[wid], idx_v)
    pltpu.sync_copy(pos_hbm, pos_v)

    @pl.loop(0, _N_CHUNKS)
    def _chunk(g):
        pltpu.async_copy(table_hbm.at[idx_v.at[g]], rows_v, gsem).wait()
        p0 = lax.rem(g * _CHUNK, _MAX_LEN)

        @pl.loop(0, _CHUNK, unroll=4)
        def _add(r):
            for j in range(_EMB // _LANES):
                pv = pos_v[p0 + r, pl.ds(j * _LANES, _LANES)]
                plsc.addupdate(rows_v.at[r, pl.ds(j * _LANES, _LANES)], pv)

        pltpu.sync_copy(rows_v, emb_out.at[pl.ds(base + g * _CHUNK, _CHUNK)])

    @pl.loop(0, _N_CHUNKS)
    def _mask(r):
        for q in range(_CHUNK // _LANES):
            x = idx_v[r, pl.ds(q * _LANES, _LANES)]
            m = jnp.where(x == 0, 0.0, 1.0).astype(jnp.float32)
            mask_v[pl.ds(r * _CHUNK + q * _LANES, _LANES)] = m

    pltpu.sync_copy(mask_v, mask_out.at[wid])


@jax.jit
def _encode(idx, table, pos):
    grid_kernel = pl.kernel(
        _body,
        out_type=[
            jax.ShapeDtypeStruct((_TOTAL, _EMB), jnp.float32),
            jax.ShapeDtypeStruct((_NW, _PER_W), jnp.float32),
        ],
        mesh=plsc.VectorSubcoreMesh(core_axis_name="c", subcore_axis_name="s"),
        scratch_types=[
            pltpu.VMEM((_N_CHUNKS, _CHUNK), jnp.int32),
            pltpu.VMEM((2 * _MAX_LEN, _EMB), jnp.float32),
            pltpu.VMEM((_CHUNK, _EMB), jnp.float32),
            pltpu.VMEM((_PER_W,), jnp.float32),
            pltpu.SemaphoreType.DMA,
        ],
    )
    return grid_kernel(idx, table, pos)


def kernel(inputs, table):
    idx = inputs.astype(jnp.int32).reshape(_NW, _N_CHUNKS, _CHUNK)
    pos = jnp.asarray(_POS2X)
    emb_flat, mask_flat = _encode(idx, table, pos)
    emb = emb_flat.reshape(_BATCH, _MAX_LEN, _EMB)
    mask = mask_flat.reshape(_BATCH, 1, _MAX_LEN)
    return emb, mask


# speedup vs baseline: 2.3913x; 2.3913x over previous
import functools

import numpy as np
import jax
import jax.numpy as jnp
from jax import lax
from jax.experimental import pallas as pl
from jax.experimental.pallas import tpu as pltpu
from jax.experimental.pallas import tpu_sc as plsc

_MAX_LEN = 200
_EMB = 64
_BATCH = 4096

_NC = 2
_NS = 16
_NW = _NC * _NS
_TOTAL = _BATCH * _MAX_LEN
_PER_W = _TOTAL // _NW
_CHUNK = 128
_N_CHUNKS = _PER_W // _CHUNK
_LANES = 16


def _pos_encoding():
    pos = np.arange(_MAX_LEN).reshape(-1, 1)
    i = np.arange(_EMB / 2).reshape(1, -1)
    pe = np.empty((_MAX_LEN, _EMB))
    pe[:, 0::2] = np.sin(pos / np.power(10000, 2 * i / _EMB))
    pe[:, 1::2] = np.cos(pos / np.power(10000, 2 * i / _EMB))
    return pe.astype(np.float32)


_POS2X = np.concatenate([_pos_encoding(), _pos_encoding()], axis=0)


def _body(idx_hbm, table_hbm, pos_hbm, emb_out, mask_out,
          idx_v, pos_v, rows_v, mask_v, gsem):
    c = lax.axis_index("c")
    s = lax.axis_index("s")
    wid = s * _NC + c
    base = wid * _PER_W

    pltpu.sync_copy(idx_hbm.at[wid], idx_v)
    pltpu.sync_copy(pos_hbm, pos_v)

    @pl.loop(0, _N_CHUNKS)
    def _chunk(g):
        pltpu.async_copy(table_hbm.at[idx_v.at[g]], rows_v, gsem).wait()
        p0 = lax.rem(g * _CHUNK, _MAX_LEN)

        @pl.loop(0, _CHUNK, unroll=4)
        def _add(r):
            for j in range(_EMB // _LANES):
                pv = pos_v[p0 + r, pl.ds(j * _LANES, _LANES)]
                plsc.addupdate(rows_v.at[r, pl.ds(j * _LANES, _LANES)], pv)

        pltpu.sync_copy(rows_v, emb_out.at[pl.ds(base + g * _CHUNK, _CHUNK)])

    @pl.loop(0, _N_CHUNKS)
    def _mask(r):
        for q in range(_CHUNK // _LANES):
            x = idx_v[r, pl.ds(q * _LANES, _LANES)]
            m = jnp.where(x == 0, 0.0, 1.0).astype(jnp.float32)
            mask_v[pl.ds(r * _CHUNK + q * _LANES, _LANES)] = m

    pltpu.sync_copy(mask_v, mask_out.at[wid])


@jax.jit
def _encode(idx, table, pos):
    grid_kernel = pl.kernel(
        _body,
        out_type=[
            jax.ShapeDtypeStruct((_TOTAL, _EMB), jnp.float32),
            jax.ShapeDtypeStruct((_NW, _PER_W), jnp.float32),
        ],
        mesh=plsc.VectorSubcoreMesh(core_axis_name="c", subcore_axis_name="s"),
        compiler_params=pltpu.CompilerParams(use_tc_tiling_on_sc=False),
        scratch_types=[
            pltpu.VMEM((_N_CHUNKS, _CHUNK), jnp.int32),
            pltpu.VMEM((2 * _MAX_LEN, _EMB), jnp.float32),
            pltpu.VMEM((_CHUNK, _EMB), jnp.float32),
            pltpu.VMEM((_PER_W,), jnp.float32),
            pltpu.SemaphoreType.DMA,
        ],
    )
    return grid_kernel(idx, table, pos)


def kernel(inputs, table):
    idx = inputs.astype(jnp.int32).reshape(_NW, _N_CHUNKS, _CHUNK)
    pos = jnp.asarray(_POS2X)
    emb_flat, mask_flat = _encode(idx, table, pos)
    emb = emb_flat.reshape(_BATCH, _MAX_LEN, _EMB)
    mask = mask_flat.reshape(_BATCH, 1, _MAX_LEN)
    return emb, mask
